# baseline (speedup 1.0000x reference)
import jax
import jax.numpy as jnp
from jax.experimental import pallas as pl
from jax.experimental.pallas import tpu as pltpu

_NP = 64
_PS = 32
_N = 16384
_R = 128
_C = 128


def _fps_d2_body(xs_ref, ys_ref, zs_ref, cenx_ref, ceny_ref, cenz_ref, d2_ref):
    xs = xs_ref[0]
    ys = ys_ref[0]
    zs = zs_ref[0]
    ri = jax.lax.broadcasted_iota(jnp.int32, (_R, _C), 0)
    ci = jax.lax.broadcasted_iota(jnp.int32, (_R, _C), 1)
    li = ri * _C + ci
    rk = jax.lax.broadcasted_iota(jnp.int32, (_NP, _C), 0)

    def step(t, carry):
        dmin, cur, ccx, ccy, ccz = carry
        sel = li == cur
        cx = jnp.sum(jnp.where(sel, xs, 0.0))
        cy = jnp.sum(jnp.where(sel, ys, 0.0))
        cz = jnp.sum(jnp.where(sel, zs, 0.0))
        ccx = jnp.where(rk == t, cx, ccx)
        ccy = jnp.where(rk == t, cy, ccy)
        ccz = jnp.where(rk == t, cz, ccz)
        dx = xs - cx
        dy = ys - cy
        dz = zs - cz
        d = (dx * dx + dy * dy) + dz * dz
        dmin = jnp.minimum(dmin, d)
        m = jnp.max(dmin)
        nxt = jnp.min(jnp.where(dmin == m, li, _N))
        return dmin, nxt, ccx, ccy, ccz

    init = (
        jnp.full((_R, _C), jnp.inf, jnp.float32),
        jnp.int32(0),
        jnp.zeros((_NP, _C), jnp.float32),
        jnp.zeros((_NP, _C), jnp.float32),
        jnp.zeros((_NP, _C), jnp.float32),
    )
    _, _, ccx, ccy, ccz = jax.lax.fori_loop(0, _NP, step, init)

    cenx_ref[0] = ccx
    ceny_ref[0] = ccy
    cenz_ref[0] = ccz

    c2 = (ccx * ccx + ccy * ccy) + ccz * ccz
    bcx = ccx.astype(jnp.bfloat16).astype(jnp.float32)
    bcy = ccy.astype(jnp.bfloat16).astype(jnp.float32)
    bcz = ccz.astype(jnp.bfloat16).astype(jnp.float32)

    def knn_chunk(j, _):
        xrow = xs_ref[0, pl.ds(j, 1), :]
        yrow = ys_ref[0, pl.ds(j, 1), :]
        zrow = zs_ref[0, pl.ds(j, 1), :]
        xb = jnp.broadcast_to(xrow, (_NP, _C))
        yb = jnp.broadcast_to(yrow, (_NP, _C))
        zb = jnp.broadcast_to(zrow, (_NP, _C))
        xb16 = xb.astype(jnp.bfloat16).astype(jnp.float32)
        yb16 = yb.astype(jnp.bfloat16).astype(jnp.float32)
        zb16 = zb.astype(jnp.bfloat16).astype(jnp.float32)
        dot = (bcx * xb16 + bcy * yb16) + bcz * zb16
        p2 = (xb * xb + yb * yb) + zb * zb
        d2c = (c2 - 2.0 * dot) + p2
        d2_ref[0, :, pl.ds(j * _C, _C)] = d2c
        return 0

    jax.lax.fori_loop(0, _R, knn_chunk, 0)


def kernel(points):
    B, N, _ = points.shape
    pts = jnp.transpose(points, (0, 2, 1))
    xs = pts[:, 0, :].reshape(B, _R, _C)
    ys = pts[:, 1, :].reshape(B, _R, _C)
    zs = pts[:, 2, :].reshape(B, _R, _C)

    spec_in = pl.BlockSpec((1, _R, _C), lambda b: (b, 0, 0))
    cen_spec = pl.BlockSpec((1, _NP, _C), lambda b: (b, 0, 0))
    d2_spec = pl.BlockSpec((1, _NP, _N), lambda b: (b, 0, 0))

    cenx, ceny, cenz, d2 = pl.pallas_call(
        _fps_d2_body,
        grid=(B,),
        in_specs=[spec_in, spec_in, spec_in],
        out_specs=[cen_spec, cen_spec, cen_spec, d2_spec],
        out_shape=[
            jax.ShapeDtypeStruct((B, _NP, _C), jnp.float32),
            jax.ShapeDtypeStruct((B, _NP, _C), jnp.float32),
            jax.ShapeDtypeStruct((B, _NP, _C), jnp.float32),
            jax.ShapeDtypeStruct((B, _NP, _N), jnp.float32),
        ],
    )(xs, ys, zs)

    centers = jnp.stack([cenx[:, :, 0], ceny[:, :, 0], cenz[:, :, 0]], axis=-1)
    _, idx = jax.lax.top_k(-d2, _PS)
    return idx, centers


# speedup vs baseline: 1.0472x; 1.0472x over previous
import jax
import jax.numpy as jnp
from jax.experimental import pallas as pl
from jax.experimental.pallas import tpu as pltpu

_NP = 64
_PS = 32
_N = 16384
_B = 32


def _fps_body(xs_ref, ys_ref, zs_ref, ctx_ref, cty_ref, ctz_ref):
    xs = xs_ref[...]
    ys = ys_ref[...]
    zs = zs_ref[...]
    li = jax.lax.broadcasted_iota(jnp.int32, (_B, _N), 1)
    eye = (jax.lax.broadcasted_iota(jnp.int32, (_B, _B), 0)
           == jax.lax.broadcasted_iota(jnp.int32, (_B, _B), 1))
    kio = jax.lax.broadcasted_iota(jnp.int32, (_NP, _B), 0)

    def transpose_col(v):
        return jnp.sum(jnp.where(eye, jnp.broadcast_to(v, (_B, _B)), 0.0),
                       axis=0, keepdims=True)

    def step(t, carry):
        dmin, cur, ctx, cty, ctz = carry
        sel = li == cur
        cx = jnp.sum(jnp.where(sel, xs, 0.0), axis=1, keepdims=True)
        cy = jnp.sum(jnp.where(sel, ys, 0.0), axis=1, keepdims=True)
        cz = jnp.sum(jnp.where(sel, zs, 0.0), axis=1, keepdims=True)
        hit = kio == t
        ctx = jnp.where(hit, jnp.broadcast_to(transpose_col(cx), (_NP, _B)), ctx)
        cty = jnp.where(hit, jnp.broadcast_to(transpose_col(cy), (_NP, _B)), cty)
        ctz = jnp.where(hit, jnp.broadcast_to(transpose_col(cz), (_NP, _B)), ctz)
        dx = xs - cx
        dy = ys - cy
        dz = zs - cz
        d = (dx * dx + dy * dy) + dz * dz
        dmin = jnp.minimum(dmin, d)
        m = jnp.max(dmin, axis=1, keepdims=True)
        nxt = jnp.min(jnp.where(dmin == m, li, _N), axis=1, keepdims=True)
        return dmin, nxt, ctx, cty, ctz

    init = (
        jnp.full((_B, _N), jnp.inf, jnp.float32),
        jnp.zeros((_B, 1), jnp.int32),
        jnp.zeros((_NP, _B), jnp.float32),
        jnp.zeros((_NP, _B), jnp.float32),
        jnp.zeros((_NP, _B), jnp.float32),
    )
    _, _, ctx, cty, ctz = jax.lax.fori_loop(0, _NP, step, init)
    ctx_ref[...] = ctx
    cty_ref[...] = cty
    ctz_ref[...] = ctz


def _d2_body(xs_ref, ys_ref, zs_ref, cx_ref, cy_ref, cz_ref, d2_ref):
    b = pl.program_id(0)
    xr = xs_ref[0]
    yr = ys_ref[0]
    zr = zs_ref[0]
    lane = jax.lax.broadcasted_iota(jnp.int32, (_NP, _B), 1)
    sel = lane == b

    def col(ref):
        return jnp.sum(jnp.where(sel, ref[...], 0.0), axis=1, keepdims=True)

    acol = col(cx_ref)
    bcol = col(cy_ref)
    ccol = col(cz_ref)
    amat = jnp.concatenate([acol, bcol, ccol], axis=1)
    pmat = jnp.concatenate([xr, yr, zr], axis=0)
    dot = jax.lax.dot_general(
        amat.astype(jnp.bfloat16), pmat.astype(jnp.bfloat16),
        (((1,), (0,)), ((), ())), preferred_element_type=jnp.float32)
    c2 = (acol * acol + bcol * bcol) + ccol * ccol
    p2 = (xr * xr + yr * yr) + zr * zr
    d2_ref[0] = (c2 - 2.0 * dot) + p2


def kernel(points):
    B, N, _ = points.shape
    pts = jnp.transpose(points, (0, 2, 1))
    xs = pts[:, 0, :]
    ys = pts[:, 1, :]
    zs = pts[:, 2, :]

    cen_shape = jax.ShapeDtypeStruct((_NP, _B), jnp.float32)
    ctx, cty, ctz = pl.pallas_call(
        _fps_body,
        out_shape=[cen_shape, cen_shape, cen_shape],
    )(xs, ys, zs)

    xs3 = xs.reshape(B, 1, N)
    ys3 = ys.reshape(B, 1, N)
    zs3 = zs.reshape(B, 1, N)
    row_spec = pl.BlockSpec((1, 1, _N), lambda b: (b, 0, 0))
    cen_spec = pl.BlockSpec((_NP, _B), lambda b: (0, 0))
    d2 = pl.pallas_call(
        _d2_body,
        grid=(B,),
        in_specs=[row_spec, row_spec, row_spec, cen_spec, cen_spec, cen_spec],
        out_specs=pl.BlockSpec((1, _NP, _N), lambda b: (b, 0, 0)),
        out_shape=jax.ShapeDtypeStruct((B, _NP, _N), jnp.float32),
    )(xs3, ys3, zs3, ctx, cty, ctz)

    centers = jnp.transpose(jnp.stack([ctx, cty, ctz], axis=0), (2, 1, 0))
    _, idx = jax.lax.top_k(-d2, _PS)
    return idx, centers


# speedup vs baseline: 10.3150x; 9.8505x over previous
import functools

import jax
import jax.numpy as jnp
from jax import lax
from jax.experimental import pallas as pl
from jax.experimental.pallas import tpu as pltpu
from jax.experimental.pallas import tpu_sc as plsc

_NP = 64
_PS = 32
_N = 16384
_B = 32
_L = 16
_STRIPS = _N // _L
_CAP = 1024
_NROWS = _B * _NP
_RPW = _NROWS // 32


def _fps_body(xs_ref, ys_ref, zs_ref, ctx_ref, cty_ref, ctz_ref):
    xs = xs_ref[...]
    ys = ys_ref[...]
    zs = zs_ref[...]
    li = jax.lax.broadcasted_iota(jnp.int32, (_B, _N), 1)
    eye = (jax.lax.broadcasted_iota(jnp.int32, (_B, _B), 0)
           == jax.lax.broadcasted_iota(jnp.int32, (_B, _B), 1))
    kio = jax.lax.broadcasted_iota(jnp.int32, (_NP, _B), 0)

    def transpose_col(v):
        return jnp.sum(jnp.where(eye, jnp.broadcast_to(v, (_B, _B)), 0.0),
                       axis=0, keepdims=True)

    def step(t, carry):
        dmin, cur, ctx, cty, ctz = carry
        sel = li == cur
        cx = jnp.sum(jnp.where(sel, xs, 0.0), axis=1, keepdims=True)
        cy = jnp.sum(jnp.where(sel, ys, 0.0), axis=1, keepdims=True)
        cz = jnp.sum(jnp.where(sel, zs, 0.0), axis=1, keepdims=True)
        hit = kio == t
        ctx = jnp.where(hit, jnp.broadcast_to(transpose_col(cx), (_NP, _B)), ctx)
        cty = jnp.where(hit, jnp.broadcast_to(transpose_col(cy), (_NP, _B)), cty)
        ctz = jnp.where(hit, jnp.broadcast_to(transpose_col(cz), (_NP, _B)), ctz)
        dx = xs - cx
        dy = ys - cy
        dz = zs - cz
        d = (dx * dx + dy * dy) + dz * dz
        dmin = jnp.minimum(dmin, d)
        m = jnp.max(dmin, axis=1, keepdims=True)
        nxt = jnp.min(jnp.where(dmin == m, li, _N), axis=1, keepdims=True)
        return dmin, nxt, ctx, cty, ctz

    init = (
        jnp.full((_B, _N), jnp.inf, jnp.float32),
        jnp.zeros((_B, 1), jnp.int32),
        jnp.zeros((_NP, _B), jnp.float32),
        jnp.zeros((_NP, _B), jnp.float32),
        jnp.zeros((_NP, _B), jnp.float32),
    )
    _, _, ctx, cty, ctz = jax.lax.fori_loop(0, _NP, step, init)
    ctx_ref[...] = ctx
    cty_ref[...] = cty
    ctz_ref[...] = ctz


def _d2_body(xs_ref, ys_ref, zs_ref, cx_ref, cy_ref, cz_ref, d2_ref):
    b = pl.program_id(0)
    xr = xs_ref[0]
    yr = ys_ref[0]
    zr = zs_ref[0]
    lane = jax.lax.broadcasted_iota(jnp.int32, (_NP, _B), 1)
    sel = lane == b

    def col(ref):
        return jnp.sum(jnp.where(sel, ref[...], 0.0), axis=1, keepdims=True)

    acol = col(cx_ref)
    bcol = col(cy_ref)
    ccol = col(cz_ref)
    amat = jnp.concatenate([acol, bcol, ccol], axis=1)
    pmat = jnp.concatenate([xr, yr, zr], axis=0)
    dot = jax.lax.dot_general(
        amat.astype(jnp.bfloat16), pmat.astype(jnp.bfloat16),
        (((1,), (0,)), ((), ())), preferred_element_type=jnp.float32)
    c2 = (acol * acol + bcol * bcol) + ccol * ccol
    p2 = (xr * xr + yr * yr) + zr * zr
    d2_ref[0] = (c2 - 2.0 * dot) + p2


def _topk_body(d2_hbm, out_hbm, rowbuf, cvals, cidx, outbuf):
    wid = lax.axis_index("s") * 2 + lax.axis_index("c")
    lanes = lax.broadcasted_iota(jnp.int32, (_L,), 0)
    lane0 = lanes == 0
    inf_v = jnp.full((_L,), jnp.inf, jnp.float32)
    big_i = jnp.full((_L,), jnp.int32(2 ** 30), jnp.int32)

    def do_row(rr, _):
        row = wid * _RPW + rr
        pltpu.sync_copy(d2_hbm.at[pl.ds(row * _N, _N)], rowbuf)

        def p1(s, carry):
            mn1, mn2 = carry
            v = rowbuf[pl.ds(s * _L, _L)]
            hi = jnp.maximum(mn1, v)
            mn1 = jnp.minimum(mn1, v)
            mn2 = jnp.minimum(mn2, hi)
            return mn1, mn2

        _, mn2 = lax.fori_loop(0, _STRIPS, p1, (inf_v, inf_v))
        t_scal = jnp.max(mn2)
        t_vec = jnp.full((_L,), t_scal, jnp.float32)

        def p2(s, carry):
            cnt, base = carry
            v = rowbuf[pl.ds(s * _L, _L)]
            m = v <= t_vec
            cs = plsc.cumsum(m.astype(jnp.int32))
            pos = cnt + cs - 1
            okm = m & (pos < _CAP) & (pos >= 0)
            plsc.store_scatter(cvals, [pos], v, mask=okm)
            plsc.store_scatter(cidx, [pos], base, mask=okm)
            cnt = cnt + plsc.all_reduce_population_count(m)
            return cnt, base + _L

        cnt_vec, _ = lax.fori_loop(
            0, _STRIPS, p2, (jnp.zeros((_L,), jnp.int32), lanes))
        tailpos = cnt_vec + lanes
        plsc.store_scatter(cvals, [tailpos], inf_v, mask=tailpos < _CAP)
        cnt = jnp.max(cnt_vec)
        cnt = jnp.minimum(cnt, _CAP)
        nstrips = (cnt + _L - 1) // _L

        def emit(j, outpos):
            def scan(s, carry):
                bv, bi, bp = carry
                v = cvals[pl.ds(s * _L, _L)]
                i = cidx[pl.ds(s * _L, _L)]
                p = s * _L + lanes
                better = (v < bv) | ((v == bv) & (i < bi))
                bv = jnp.where(better, v, bv)
                bi = jnp.where(better, i, bi)
                bp = jnp.where(better, p, bp)
                return bv, bi, bp

            bv, bi, bp = lax.fori_loop(0, nstrips, scan, (inf_v, big_i, big_i))
            mval = jnp.min(bv)
            msel = bv == jnp.full((_L,), mval, jnp.float32)
            midx = jnp.min(jnp.where(msel, bi, big_i))
            midx_vec = jnp.full((_L,), midx, jnp.int32)
            wsel = msel & (bi == midx_vec)
            pw = jnp.min(jnp.where(wsel, bp, big_i))
            pw_vec = jnp.full((_L,), pw, jnp.int32)
            plsc.store_scatter(cvals, [pw_vec], inf_v, mask=lane0)
            plsc.store_scatter(outbuf, [outpos], midx_vec, mask=lane0)
            return outpos + 1

        lax.fori_loop(0, _PS, emit, rr * _PS + jnp.zeros((_L,), jnp.int32))
        return 0

    lax.fori_loop(0, _RPW, do_row, 0)
    pltpu.sync_copy(outbuf, out_hbm.at[pl.ds(wid * (_RPW * _PS), _RPW * _PS)])


def _sc_topk(d2_flat):
    mesh = plsc.VectorSubcoreMesh(core_axis_name="c", subcore_axis_name="s")
    kfn = functools.partial(
        pl.kernel,
        out_type=jax.ShapeDtypeStruct((_NROWS * _PS,), jnp.int32),
        mesh=mesh,
        scratch_types=[
            pltpu.VMEM((_N,), jnp.float32),
            pltpu.VMEM((_CAP,), jnp.float32),
            pltpu.VMEM((_CAP,), jnp.int32),
            pltpu.VMEM((_RPW * _PS,), jnp.int32),
        ],
        compiler_params=pltpu.CompilerParams(needs_layout_passes=False),
    )(_topk_body)
    return kfn(d2_flat)


def kernel(points):
    B, N, _ = points.shape
    pts = jnp.transpose(points, (0, 2, 1))
    xs = pts[:, 0, :]
    ys = pts[:, 1, :]
    zs = pts[:, 2, :]

    cen_shape = jax.ShapeDtypeStruct((_NP, _B), jnp.float32)
    ctx, cty, ctz = pl.pallas_call(
        _fps_body,
        out_shape=[cen_shape, cen_shape, cen_shape],
    )(xs, ys, zs)

    xs3 = xs.reshape(B, 1, N)
    ys3 = ys.reshape(B, 1, N)
    zs3 = zs.reshape(B, 1, N)
    row_spec = pl.BlockSpec((1, 1, _N), lambda b: (b, 0, 0))
    cen_spec = pl.BlockSpec((_NP, _B), lambda b: (0, 0))
    d2 = pl.pallas_call(
        _d2_body,
        grid=(B,),
        in_specs=[row_spec, row_spec, row_spec, cen_spec, cen_spec, cen_spec],
        out_specs=pl.BlockSpec((1, _NP, _N), lambda b: (b, 0, 0)),
        out_shape=jax.ShapeDtypeStruct((B, _NP, _N), jnp.float32),
    )(xs3, ys3, zs3, ctx, cty, ctz)

    centers = jnp.transpose(jnp.stack([ctx, cty, ctz], axis=0), (2, 1, 0))
    idx = _sc_topk(d2.reshape(-1)).reshape(B, _NP, _PS)
    return idx, centers


# speedup vs baseline: 12.6786x; 1.2291x over previous
import functools

import jax
import jax.numpy as jnp
from jax import lax
from jax.experimental import pallas as pl
from jax.experimental.pallas import tpu as pltpu
from jax.experimental.pallas import tpu_sc as plsc

_NP = 64
_PS = 32
_N = 16384
_B = 32
_L = 16
_STRIPS = _N // _L
_CAP = 1024
_NROWS = _B * _NP
_RPW = _NROWS // 32


def _fps_body(xs_ref, ys_ref, zs_ref, ctx_ref, cty_ref, ctz_ref):
    xs = xs_ref[...]
    ys = ys_ref[...]
    zs = zs_ref[...]
    li = jax.lax.broadcasted_iota(jnp.int32, (_B, _N), 1)
    eye = (jax.lax.broadcasted_iota(jnp.int32, (_B, _B), 0)
           == jax.lax.broadcasted_iota(jnp.int32, (_B, _B), 1))
    kio = jax.lax.broadcasted_iota(jnp.int32, (_NP, _B), 0)

    def transpose_col(v):
        return jnp.sum(jnp.where(eye, jnp.broadcast_to(v, (_B, _B)), 0.0),
                       axis=0, keepdims=True)

    def step(t, carry):
        dmin, cur, ctx, cty, ctz = carry
        sel = li == cur
        cx = jnp.sum(jnp.where(sel, xs, 0.0), axis=1, keepdims=True)
        cy = jnp.sum(jnp.where(sel, ys, 0.0), axis=1, keepdims=True)
        cz = jnp.sum(jnp.where(sel, zs, 0.0), axis=1, keepdims=True)
        hit = kio == t
        ctx = jnp.where(hit, jnp.broadcast_to(transpose_col(cx), (_NP, _B)), ctx)
        cty = jnp.where(hit, jnp.broadcast_to(transpose_col(cy), (_NP, _B)), cty)
        ctz = jnp.where(hit, jnp.broadcast_to(transpose_col(cz), (_NP, _B)), ctz)
        dx = xs - cx
        dy = ys - cy
        dz = zs - cz
        d = (dx * dx + dy * dy) + dz * dz
        dmin = jnp.minimum(dmin, d)
        m = jnp.max(dmin, axis=1, keepdims=True)
        nxt = jnp.min(jnp.where(dmin == m, li, _N), axis=1, keepdims=True)
        return dmin, nxt, ctx, cty, ctz

    init = (
        jnp.full((_B, _N), jnp.inf, jnp.float32),
        jnp.zeros((_B, 1), jnp.int32),
        jnp.zeros((_NP, _B), jnp.float32),
        jnp.zeros((_NP, _B), jnp.float32),
        jnp.zeros((_NP, _B), jnp.float32),
    )
    _, _, ctx, cty, ctz = jax.lax.fori_loop(0, _NP, step, init)
    ctx_ref[...] = ctx
    cty_ref[...] = cty
    ctz_ref[...] = ctz


def _d2_body(xs_ref, ys_ref, zs_ref, cx_ref, cy_ref, cz_ref, d2_ref):
    b = pl.program_id(0)
    xr = xs_ref[0]
    yr = ys_ref[0]
    zr = zs_ref[0]
    lane = jax.lax.broadcasted_iota(jnp.int32, (_NP, _B), 1)
    sel = lane == b

    def col(ref):
        return jnp.sum(jnp.where(sel, ref[...], 0.0), axis=1, keepdims=True)

    acol = col(cx_ref)
    bcol = col(cy_ref)
    ccol = col(cz_ref)
    amat = jnp.concatenate([acol, bcol, ccol], axis=1)
    pmat = jnp.concatenate([xr, yr, zr], axis=0)
    dot = jax.lax.dot_general(
        amat.astype(jnp.bfloat16), pmat.astype(jnp.bfloat16),
        (((1,), (0,)), ((), ())), preferred_element_type=jnp.float32)
    c2 = (acol * acol + bcol * bcol) + ccol * ccol
    p2 = (xr * xr + yr * yr) + zr * zr
    d2_ref[0] = (c2 - 2.0 * dot) + p2


def _topk_body(d2_hbm, out_hbm, rowbuf0, rowbuf1, cvals, cidx, outbuf,
               sem0, sem1):
    wid = lax.axis_index("s") * 2 + lax.axis_index("c")
    lanes = lax.broadcasted_iota(jnp.int32, (_L,), 0)
    lane0 = lanes == 0
    inf_v = jnp.full((_L,), jnp.inf, jnp.float32)
    big_i = jnp.full((_L,), jnp.int32(2 ** 30), jnp.int32)
    base_row = wid * _RPW

    def row_src(rr):
        r = jnp.minimum(base_row + rr, _NROWS - 1)
        return d2_hbm.at[pl.ds(r * _N, _N)]

    def process_row(rr, rowbuf):
        def p1(s, carry):
            mn1, mn2 = carry
            v = rowbuf[pl.ds(s * _L, _L)]
            hi = jnp.maximum(mn1, v)
            mn1 = jnp.minimum(mn1, v)
            mn2 = jnp.minimum(mn2, hi)
            return mn1, mn2

        _, mn2 = lax.fori_loop(0, _STRIPS, p1, (inf_v, inf_v), unroll=8)
        t_scal = jnp.max(mn2)
        t_vec = jnp.full((_L,), t_scal, jnp.float32)

        def p2(s, carry):
            cnt, base = carry
            v = rowbuf[pl.ds(s * _L, _L)]
            m = v <= t_vec
            cs = plsc.cumsum(m.astype(jnp.int32))
            pos = cnt + cs - 1
            okm = m & (pos < _CAP)
            plsc.store_scatter(cvals, [pos], v, mask=okm)
            plsc.store_scatter(cidx, [pos], base, mask=okm)
            cnt = cnt + plsc.all_reduce_population_count(m)
            return cnt, base + _L

        cnt_vec, _ = lax.fori_loop(
            0, _STRIPS, p2, (jnp.zeros((_L,), jnp.int32), lanes), unroll=8)
        tailpos = cnt_vec + lanes
        plsc.store_scatter(cvals, [tailpos], inf_v, mask=tailpos < _CAP)
        cnt = jnp.max(cnt_vec)
        cnt = jnp.minimum(cnt, _CAP)
        nstrips = (cnt + _L - 1) // _L

        def emit(j, outpos):
            def scan(s, carry):
                bv, bi, bp = carry
                v = cvals[pl.ds(s * _L, _L)]
                i = cidx[pl.ds(s * _L, _L)]
                p = s * _L + lanes
                better = (v < bv) | ((v == bv) & (i < bi))
                bv = jnp.where(better, v, bv)
                bi = jnp.where(better, i, bi)
                bp = jnp.where(better, p, bp)
                return bv, bi, bp

            bv, bi, bp = lax.fori_loop(0, nstrips, scan, (inf_v, big_i, big_i))
            mval = jnp.min(bv)
            msel = bv == jnp.full((_L,), mval, jnp.float32)
            midx = jnp.min(jnp.where(msel, bi, big_i))
            midx_vec = jnp.full((_L,), midx, jnp.int32)
            wsel = msel & (bi == midx_vec)
            pw = jnp.min(jnp.where(wsel, bp, big_i))
            pw_vec = jnp.full((_L,), pw, jnp.int32)
            plsc.store_scatter(cvals, [pw_vec], inf_v, mask=lane0)
            plsc.store_scatter(outbuf, [outpos], midx_vec, mask=lane0)
            return outpos + 1

        lax.fori_loop(0, _PS, emit, rr * _PS + jnp.zeros((_L,), jnp.int32))

    pltpu.async_copy(row_src(0), rowbuf0, sem0)
    pltpu.async_copy(row_src(1), rowbuf1, sem1)

    def do_pair(i, _):
        rr = 2 * i
        pltpu.make_async_copy(row_src(rr), rowbuf0, sem0).wait()
        process_row(rr, rowbuf0)
        pltpu.async_copy(row_src(rr + 2), rowbuf0, sem0)
        pltpu.make_async_copy(row_src(rr + 1), rowbuf1, sem1).wait()
        process_row(rr + 1, rowbuf1)
        pltpu.async_copy(row_src(rr + 3), rowbuf1, sem1)
        return 0

    lax.fori_loop(0, _RPW // 2, do_pair, 0)
    pltpu.make_async_copy(row_src(0), rowbuf0, sem0).wait()
    pltpu.make_async_copy(row_src(1), rowbuf1, sem1).wait()
    pltpu.sync_copy(outbuf, out_hbm.at[pl.ds(wid * (_RPW * _PS), _RPW * _PS)])


def _sc_topk(d2_flat):
    mesh = plsc.VectorSubcoreMesh(core_axis_name="c", subcore_axis_name="s")
    kfn = functools.partial(
        pl.kernel,
        out_type=jax.ShapeDtypeStruct((_NROWS * _PS,), jnp.int32),
        mesh=mesh,
        scratch_types=[
            pltpu.VMEM((_N,), jnp.float32),
            pltpu.VMEM((_N,), jnp.float32),
            pltpu.VMEM((_CAP,), jnp.float32),
            pltpu.VMEM((_CAP,), jnp.int32),
            pltpu.VMEM((_RPW * _PS,), jnp.int32),
            pltpu.SemaphoreType.DMA,
            pltpu.SemaphoreType.DMA,
        ],
        compiler_params=pltpu.CompilerParams(needs_layout_passes=False),
    )(_topk_body)
    return kfn(d2_flat)


def kernel(points):
    B, N, _ = points.shape
    pts = jnp.transpose(points, (0, 2, 1))
    xs = pts[:, 0, :]
    ys = pts[:, 1, :]
    zs = pts[:, 2, :]

    cen_shape = jax.ShapeDtypeStruct((_NP, _B), jnp.float32)
    ctx, cty, ctz = pl.pallas_call(
        _fps_body,
        out_shape=[cen_shape, cen_shape, cen_shape],
    )(xs, ys, zs)

    xs3 = xs.reshape(B, 1, N)
    ys3 = ys.reshape(B, 1, N)
    zs3 = zs.reshape(B, 1, N)
    row_spec = pl.BlockSpec((1, 1, _N), lambda b: (b, 0, 0))
    cen_spec = pl.BlockSpec((_NP, _B), lambda b: (0, 0))
    d2 = pl.pallas_call(
        _d2_body,
        grid=(B,),
        in_specs=[row_spec, row_spec, row_spec, cen_spec, cen_spec, cen_spec],
        out_specs=pl.BlockSpec((1, _NP, _N), lambda b: (b, 0, 0)),
        out_shape=jax.ShapeDtypeStruct((B, _NP, _N), jnp.float32),
    )(xs3, ys3, zs3, ctx, cty, ctz)

    centers = jnp.transpose(jnp.stack([ctx, cty, ctz], axis=0), (2, 1, 0))
    idx = _sc_topk(d2.reshape(-1)).reshape(B, _NP, _PS)
    return idx, centers


# speedup vs baseline: 16.9486x; 1.3368x over previous
import functools

import jax
import jax.numpy as jnp
from jax import lax
from jax.experimental import pallas as pl
from jax.experimental.pallas import tpu as pltpu
from jax.experimental.pallas import tpu_sc as plsc

_NP = 64
_PS = 32
_N = 16384
_B = 32
_L = 16
_STRIPS = _N // _L
_CAP = 1024
_NROWS = _B * _NP
_RPW = _NROWS // 32


def _fps_body(xs_ref, ys_ref, zs_ref, ctx_ref, cty_ref, ctz_ref):
    xs = xs_ref[...]
    ys = ys_ref[...]
    zs = zs_ref[...]
    li = jax.lax.broadcasted_iota(jnp.int32, (_B, _N), 1)
    eye = (jax.lax.broadcasted_iota(jnp.int32, (_B, _B), 0)
           == jax.lax.broadcasted_iota(jnp.int32, (_B, _B), 1))
    kio = jax.lax.broadcasted_iota(jnp.int32, (_NP, _B), 0)

    def transpose_col(v):
        return jnp.sum(jnp.where(eye, jnp.broadcast_to(v, (_B, _B)), 0.0),
                       axis=0, keepdims=True)

    def step(t, carry):
        dmin, cur, ctx, cty, ctz = carry
        sel = li == cur
        cx = jnp.sum(jnp.where(sel, xs, 0.0), axis=1, keepdims=True)
        cy = jnp.sum(jnp.where(sel, ys, 0.0), axis=1, keepdims=True)
        cz = jnp.sum(jnp.where(sel, zs, 0.0), axis=1, keepdims=True)
        hit = kio == t
        ctx = jnp.where(hit, jnp.broadcast_to(transpose_col(cx), (_NP, _B)), ctx)
        cty = jnp.where(hit, jnp.broadcast_to(transpose_col(cy), (_NP, _B)), cty)
        ctz = jnp.where(hit, jnp.broadcast_to(transpose_col(cz), (_NP, _B)), ctz)
        dx = xs - cx
        dy = ys - cy
        dz = zs - cz
        d = (dx * dx + dy * dy) + dz * dz
        dmin = jnp.minimum(dmin, d)
        m = jnp.max(dmin, axis=1, keepdims=True)
        nxt = jnp.min(jnp.where(dmin == m, li, _N), axis=1, keepdims=True)
        return dmin, nxt, ctx, cty, ctz

    init = (
        jnp.full((_B, _N), jnp.inf, jnp.float32),
        jnp.zeros((_B, 1), jnp.int32),
        jnp.zeros((_NP, _B), jnp.float32),
        jnp.zeros((_NP, _B), jnp.float32),
        jnp.zeros((_NP, _B), jnp.float32),
    )
    _, _, ctx, cty, ctz = jax.lax.fori_loop(0, _NP, step, init)
    ctx_ref[...] = ctx
    cty_ref[...] = cty
    ctz_ref[...] = ctz


def _d2_body(xs_ref, ys_ref, zs_ref, cx_ref, cy_ref, cz_ref, d2_ref):
    b = pl.program_id(0)
    xr = xs_ref[0]
    yr = ys_ref[0]
    zr = zs_ref[0]
    lane = jax.lax.broadcasted_iota(jnp.int32, (_NP, _B), 1)
    sel = lane == b

    def col(ref):
        return jnp.sum(jnp.where(sel, ref[...], 0.0), axis=1, keepdims=True)

    acol = col(cx_ref)
    bcol = col(cy_ref)
    ccol = col(cz_ref)
    amat = jnp.concatenate([acol, bcol, ccol], axis=1)
    pmat = jnp.concatenate([xr, yr, zr], axis=0)
    dot = jax.lax.dot_general(
        amat.astype(jnp.bfloat16), pmat.astype(jnp.bfloat16),
        (((1,), (0,)), ((), ())), preferred_element_type=jnp.float32)
    c2 = (acol * acol + bcol * bcol) + ccol * ccol
    p2 = (xr * xr + yr * yr) + zr * zr
    d2_ref[0] = (c2 - 2.0 * dot) + p2


def _topk_body(d2_hbm, out_hbm, rowbuf0, rowbuf1, cvals, cidx, outbuf,
               sem0, sem1):
    wid = lax.axis_index("s") * 2 + lax.axis_index("c")
    lanes = lax.broadcasted_iota(jnp.int32, (_L,), 0)
    lane0 = lanes == 0
    inf_v = jnp.full((_L,), jnp.inf, jnp.float32)
    big_i = jnp.full((_L,), jnp.int32(2 ** 30), jnp.int32)
    base_row = wid * _RPW

    def row_src(rr):
        r = jnp.minimum(base_row + rr, _NROWS - 1)
        return d2_hbm.at[pl.ds(r * _N, _N)]

    def process_row(rr, rowbuf):
        def p1(s, carry):
            mn1, mn2 = carry
            v = rowbuf[pl.ds(s * _L, _L)]
            hi = jnp.maximum(mn1, v)
            mn1 = jnp.minimum(mn1, v)
            mn2 = jnp.minimum(mn2, hi)
            return mn1, mn2

        _, mn2 = lax.fori_loop(0, _STRIPS, p1, (inf_v, inf_v), unroll=8)
        t_scal = jnp.max(mn2)
        t_vec = jnp.full((_L,), t_scal, jnp.float32)

        for t in range(_CAP // _L):
            cvals[pl.ds(t * _L, _L)] = inf_v

        def p2(s, carry):
            pos_vec, base = carry
            v = rowbuf[pl.ds(s * _L, _L)]
            m = v <= t_vec
            okm = m & (pos_vec < _CAP)
            plsc.store_scatter(cvals, [pos_vec], v, mask=okm)
            plsc.store_scatter(cidx, [pos_vec], base, mask=okm)
            pos_vec = pos_vec + (m.astype(jnp.int32) << 4)
            return pos_vec, base + _L

        pos_vec, _ = lax.fori_loop(0, _STRIPS, p2, (lanes, lanes), unroll=8)
        nstrips = jnp.minimum(jnp.max((pos_vec - lanes) >> 4), _CAP // _L)

        def emit(j, outpos):
            def scan(s, carry):
                bv, bi, bp = carry
                v = cvals[pl.ds(s * _L, _L)]
                i = cidx[pl.ds(s * _L, _L)]
                p = s * _L + lanes
                better = (v < bv) | ((v == bv) & (i < bi))
                bv = jnp.where(better, v, bv)
                bi = jnp.where(better, i, bi)
                bp = jnp.where(better, p, bp)
                return bv, bi, bp

            bv, bi, bp = lax.fori_loop(0, nstrips, scan, (inf_v, big_i, big_i))
            mval = jnp.min(bv)
            msel = bv == jnp.full((_L,), mval, jnp.float32)
            midx = jnp.min(jnp.where(msel, bi, big_i))
            midx_vec = jnp.full((_L,), midx, jnp.int32)
            wsel = msel & (bi == midx_vec)
            pw = jnp.min(jnp.where(wsel, bp, big_i))
            pw_vec = jnp.full((_L,), pw, jnp.int32)
            plsc.store_scatter(cvals, [pw_vec], inf_v, mask=lane0)
            plsc.store_scatter(outbuf, [outpos], midx_vec, mask=lane0)
            return outpos + 1

        lax.fori_loop(0, _PS, emit, rr * _PS + jnp.zeros((_L,), jnp.int32))

    pltpu.async_copy(row_src(0), rowbuf0, sem0)
    pltpu.async_copy(row_src(1), rowbuf1, sem1)

    def do_pair(i, _):
        rr = 2 * i
        pltpu.make_async_copy(row_src(rr), rowbuf0, sem0).wait()
        process_row(rr, rowbuf0)
        pltpu.async_copy(row_src(rr + 2), rowbuf0, sem0)
        pltpu.make_async_copy(row_src(rr + 1), rowbuf1, sem1).wait()
        process_row(rr + 1, rowbuf1)
        pltpu.async_copy(row_src(rr + 3), rowbuf1, sem1)
        return 0

    lax.fori_loop(0, _RPW // 2, do_pair, 0)
    pltpu.make_async_copy(row_src(0), rowbuf0, sem0).wait()
    pltpu.make_async_copy(row_src(1), rowbuf1, sem1).wait()
    pltpu.sync_copy(outbuf, out_hbm.at[pl.ds(wid * (_RPW * _PS), _RPW * _PS)])


def _sc_topk(d2_flat):
    mesh = plsc.VectorSubcoreMesh(core_axis_name="c", subcore_axis_name="s")
    kfn = functools.partial(
        pl.kernel,
        out_type=jax.ShapeDtypeStruct((_NROWS * _PS,), jnp.int32),
        mesh=mesh,
        scratch_types=[
            pltpu.VMEM((_N,), jnp.float32),
            pltpu.VMEM((_N,), jnp.float32),
            pltpu.VMEM((_CAP,), jnp.float32),
            pltpu.VMEM((_CAP,), jnp.int32),
            pltpu.VMEM((_RPW * _PS,), jnp.int32),
            pltpu.SemaphoreType.DMA,
            pltpu.SemaphoreType.DMA,
        ],
        compiler_params=pltpu.CompilerParams(needs_layout_passes=False),
    )(_topk_body)
    return kfn(d2_flat)


def kernel(points):
    B, N, _ = points.shape
    pts = jnp.transpose(points, (0, 2, 1))
    xs = pts[:, 0, :]
    ys = pts[:, 1, :]
    zs = pts[:, 2, :]

    cen_shape = jax.ShapeDtypeStruct((_NP, _B), jnp.float32)
    ctx, cty, ctz = pl.pallas_call(
        _fps_body,
        out_shape=[cen_shape, cen_shape, cen_shape],
    )(xs, ys, zs)

    xs3 = xs.reshape(B, 1, N)
    ys3 = ys.reshape(B, 1, N)
    zs3 = zs.reshape(B, 1, N)
    row_spec = pl.BlockSpec((1, 1, _N), lambda b: (b, 0, 0))
    cen_spec = pl.BlockSpec((_NP, _B), lambda b: (0, 0))
    d2 = pl.pallas_call(
        _d2_body,
        grid=(B,),
        in_specs=[row_spec, row_spec, row_spec, cen_spec, cen_spec, cen_spec],
        out_specs=pl.BlockSpec((1, _NP, _N), lambda b: (b, 0, 0)),
        out_shape=jax.ShapeDtypeStruct((B, _NP, _N), jnp.float32),
    )(xs3, ys3, zs3, ctx, cty, ctz)

    centers = jnp.transpose(jnp.stack([ctx, cty, ctz], axis=0), (2, 1, 0))
    idx = _sc_topk(d2.reshape(-1)).reshape(B, _NP, _PS)
    return idx, centers


# speedup vs baseline: 17.0727x; 1.0073x over previous
import functools

import jax
import jax.numpy as jnp
from jax import lax
from jax.experimental import pallas as pl
from jax.experimental.pallas import tpu as pltpu
from jax.experimental.pallas import tpu_sc as plsc

_NP = 64
_PS = 32
_N = 16384
_B = 32
_L = 16
_STRIPS = _N // _L
_CAP = 1024
_NROWS = _B * _NP
_RPW = _NROWS // 32


def _fps_body(xs_ref, ys_ref, zs_ref, ctx_ref, cty_ref, ctz_ref):
    xs = xs_ref[...]
    ys = ys_ref[...]
    zs = zs_ref[...]
    li = jax.lax.broadcasted_iota(jnp.int32, (_B, _N), 1)
    eye = (jax.lax.broadcasted_iota(jnp.int32, (_B, _B), 0)
           == jax.lax.broadcasted_iota(jnp.int32, (_B, _B), 1))
    kio = jax.lax.broadcasted_iota(jnp.int32, (_NP, _B), 0)

    def transpose_col(v):
        return jnp.sum(jnp.where(eye, jnp.broadcast_to(v, (_B, _B)), 0.0),
                       axis=0, keepdims=True)

    def step(t, carry):
        dmin, cur, ctx, cty, ctz = carry
        sel = li == cur
        cx = jnp.sum(jnp.where(sel, xs, 0.0), axis=1, keepdims=True)
        cy = jnp.sum(jnp.where(sel, ys, 0.0), axis=1, keepdims=True)
        cz = jnp.sum(jnp.where(sel, zs, 0.0), axis=1, keepdims=True)
        hit = kio == t
        ctx = jnp.where(hit, jnp.broadcast_to(transpose_col(cx), (_NP, _B)), ctx)
        cty = jnp.where(hit, jnp.broadcast_to(transpose_col(cy), (_NP, _B)), cty)
        ctz = jnp.where(hit, jnp.broadcast_to(transpose_col(cz), (_NP, _B)), ctz)
        dx = xs - cx
        dy = ys - cy
        dz = zs - cz
        d = (dx * dx + dy * dy) + dz * dz
        dmin = jnp.minimum(dmin, d)
        m = jnp.max(dmin, axis=1, keepdims=True)
        nxt = jnp.min(jnp.where(dmin == m, li, _N), axis=1, keepdims=True)
        return dmin, nxt, ctx, cty, ctz

    init = (
        jnp.full((_B, _N), jnp.inf, jnp.float32),
        jnp.zeros((_B, 1), jnp.int32),
        jnp.zeros((_NP, _B), jnp.float32),
        jnp.zeros((_NP, _B), jnp.float32),
        jnp.zeros((_NP, _B), jnp.float32),
    )
    _, _, ctx, cty, ctz = jax.lax.fori_loop(0, _NP, step, init)
    ctx_ref[...] = ctx
    cty_ref[...] = cty
    ctz_ref[...] = ctz


def _d2_body(xs_ref, ys_ref, zs_ref, cx_ref, cy_ref, cz_ref, d2_ref):
    b = pl.program_id(0)
    xr = xs_ref[0]
    yr = ys_ref[0]
    zr = zs_ref[0]
    lane = jax.lax.broadcasted_iota(jnp.int32, (_NP, _B), 1)
    sel = lane == b

    def col(ref):
        return jnp.sum(jnp.where(sel, ref[...], 0.0), axis=1, keepdims=True)

    acol = col(cx_ref)
    bcol = col(cy_ref)
    ccol = col(cz_ref)
    amat = jnp.concatenate([acol, bcol, ccol], axis=1)
    pmat = jnp.concatenate([xr, yr, zr], axis=0)
    dot = jax.lax.dot_general(
        amat.astype(jnp.bfloat16), pmat.astype(jnp.bfloat16),
        (((1,), (0,)), ((), ())), preferred_element_type=jnp.float32)
    c2 = (acol * acol + bcol * bcol) + ccol * ccol
    p2 = (xr * xr + yr * yr) + zr * zr
    d2_ref[0] = (c2 - 2.0 * dot) + p2


def _topk_body(d2_hbm, out_hbm, rowbuf0, rowbuf1, cvals, cidx, outbuf,
               sem0, sem1):
    wid = lax.axis_index("s") * 2 + lax.axis_index("c")
    lanes = lax.broadcasted_iota(jnp.int32, (_L,), 0)
    lane0 = lanes == 0
    inf_v = jnp.full((_L,), jnp.inf, jnp.float32)
    big_i = jnp.full((_L,), jnp.int32(2 ** 30), jnp.int32)
    base_row = wid * _RPW

    def row_src(rr):
        r = jnp.minimum(base_row + rr, _NROWS - 1)
        return d2_hbm.at[pl.ds(r * _N, _N)]

    def process_row(rr, rowbuf):
        def p1(s, carry):
            mn1, mn2 = carry
            v = rowbuf[pl.ds(s * _L, _L)]
            hi = jnp.maximum(mn1, v)
            mn1 = jnp.minimum(mn1, v)
            mn2 = jnp.minimum(mn2, hi)
            return mn1, mn2

        _, mn2 = lax.fori_loop(0, _STRIPS, p1, (inf_v, inf_v), unroll=8)
        t_scal = jnp.max(mn2)
        t_vec = jnp.full((_L,), t_scal, jnp.float32)

        for t in range(_CAP // _L):
            cvals[pl.ds(t * _L, _L)] = inf_v

        def p2(s, carry):
            pos_vec, base = carry
            v = rowbuf[pl.ds(s * _L, _L)]
            m = v <= t_vec
            okm = m & (pos_vec < _CAP)
            plsc.store_scatter(cvals, [pos_vec], v, mask=okm)
            plsc.store_scatter(cidx, [pos_vec], base, mask=okm)
            pos_vec = pos_vec + (m.astype(jnp.int32) << 4)
            return pos_vec, base + _L

        pos_vec, _ = lax.fori_loop(0, _STRIPS, p2, (lanes, lanes), unroll=8)
        nstrips = jnp.minimum(jnp.max((pos_vec - lanes) >> 4), _CAP // _L)

        def mk_emit(nscan, unroll):
            def emit(j, outpos):
                def scan(s, carry):
                    bv, bi, bp = carry
                    v = cvals[pl.ds(s * _L, _L)]
                    i = cidx[pl.ds(s * _L, _L)]
                    p = s * _L + lanes
                    better = (v < bv) | ((v == bv) & (i < bi))
                    bv = jnp.where(better, v, bv)
                    bi = jnp.where(better, i, bi)
                    bp = jnp.where(better, p, bp)
                    return bv, bi, bp

                bv, bi, bp = lax.fori_loop(
                    0, nscan, scan, (inf_v, big_i, big_i), unroll=unroll)
                mval = jnp.min(bv)
                msel = bv == jnp.full((_L,), mval, jnp.float32)
                midx = jnp.min(jnp.where(msel, bi, big_i))
                midx_vec = jnp.full((_L,), midx, jnp.int32)
                wsel = msel & (bi == midx_vec)
                pw = jnp.min(jnp.where(wsel, bp, big_i))
                pw_vec = jnp.full((_L,), pw, jnp.int32)
                plsc.store_scatter(cvals, [pw_vec], inf_v, mask=lane0)
                plsc.store_scatter(outbuf, [outpos], midx_vec, mask=lane0)
                return outpos + 1

            return emit

        out0 = rr * _PS + jnp.zeros((_L,), jnp.int32)

        def fast_sel():
            lax.fori_loop(0, _PS, mk_emit(16, 4), out0)
            return 0

        def slow_sel():
            lax.fori_loop(0, _PS, mk_emit(nstrips, 1), out0)
            return 0

        lax.cond(nstrips <= 16, fast_sel, slow_sel)

    pltpu.async_copy(row_src(0), rowbuf0, sem0)
    pltpu.async_copy(row_src(1), rowbuf1, sem1)

    def do_pair(i, _):
        rr = 2 * i
        pltpu.make_async_copy(row_src(rr), rowbuf0, sem0).wait()
        process_row(rr, rowbuf0)
        pltpu.async_copy(row_src(rr + 2), rowbuf0, sem0)
        pltpu.make_async_copy(row_src(rr + 1), rowbuf1, sem1).wait()
        process_row(rr + 1, rowbuf1)
        pltpu.async_copy(row_src(rr + 3), rowbuf1, sem1)
        return 0

    lax.fori_loop(0, _RPW // 2, do_pair, 0)
    pltpu.make_async_copy(row_src(0), rowbuf0, sem0).wait()
    pltpu.make_async_copy(row_src(1), rowbuf1, sem1).wait()
    pltpu.sync_copy(outbuf, out_hbm.at[pl.ds(wid * (_RPW * _PS), _RPW * _PS)])


def _sc_topk(d2_flat):
    mesh = plsc.VectorSubcoreMesh(core_axis_name="c", subcore_axis_name="s")
    kfn = functools.partial(
        pl.kernel,
        out_type=jax.ShapeDtypeStruct((_NROWS * _PS,), jnp.int32),
        mesh=mesh,
        scratch_types=[
            pltpu.VMEM((_N,), jnp.float32),
            pltpu.VMEM((_N,), jnp.float32),
            pltpu.VMEM((_CAP,), jnp.float32),
            pltpu.VMEM((_CAP,), jnp.int32),
            pltpu.VMEM((_RPW * _PS,), jnp.int32),
            pltpu.SemaphoreType.DMA,
            pltpu.SemaphoreType.DMA,
        ],
        compiler_params=pltpu.CompilerParams(needs_layout_passes=False),
    )(_topk_body)
    return kfn(d2_flat)


def kernel(points):
    B, N, _ = points.shape
    pts = jnp.transpose(points, (0, 2, 1))
    xs = pts[:, 0, :]
    ys = pts[:, 1, :]
    zs = pts[:, 2, :]

    cen_shape = jax.ShapeDtypeStruct((_NP, _B), jnp.float32)
    ctx, cty, ctz = pl.pallas_call(
        _fps_body,
        out_shape=[cen_shape, cen_shape, cen_shape],
    )(xs, ys, zs)

    xs3 = xs.reshape(B, 1, N)
    ys3 = ys.reshape(B, 1, N)
    zs3 = zs.reshape(B, 1, N)
    row_spec = pl.BlockSpec((1, 1, _N), lambda b: (b, 0, 0))
    cen_spec = pl.BlockSpec((_NP, _B), lambda b: (0, 0))
    d2 = pl.pallas_call(
        _d2_body,
        grid=(B,),
        in_specs=[row_spec, row_spec, row_spec, cen_spec, cen_spec, cen_spec],
        out_specs=pl.BlockSpec((1, _NP, _N), lambda b: (b, 0, 0)),
        out_shape=jax.ShapeDtypeStruct((B, _NP, _N), jnp.float32),
    )(xs3, ys3, zs3, ctx, cty, ctz)

    centers = jnp.transpose(jnp.stack([ctx, cty, ctz], axis=0), (2, 1, 0))
    idx = _sc_topk(d2.reshape(-1)).reshape(B, _NP, _PS)
    return idx, centers


# speedup vs baseline: 17.1070x; 1.0020x over previous
import functools

import jax
import jax.numpy as jnp
from jax import lax
from jax.experimental import pallas as pl
from jax.experimental.pallas import tpu as pltpu
from jax.experimental.pallas import tpu_sc as plsc

_NP = 64
_PS = 32
_N = 16384
_B = 32
_L = 16
_STRIPS = _N // _L
_CAP = 1024
_NROWS = _B * _NP
_RPW = _NROWS // 32


def _fps_body(xs_ref, ys_ref, zs_ref, ctx_ref, cty_ref, ctz_ref):
    xs = xs_ref[...]
    ys = ys_ref[...]
    zs = zs_ref[...]
    li = jax.lax.broadcasted_iota(jnp.int32, (_B, _N), 1)
    eye = (jax.lax.broadcasted_iota(jnp.int32, (_B, _B), 0)
           == jax.lax.broadcasted_iota(jnp.int32, (_B, _B), 1))
    kio = jax.lax.broadcasted_iota(jnp.int32, (_NP, _B), 0)

    def transpose_col(v):
        return jnp.sum(jnp.where(eye, jnp.broadcast_to(v, (_B, _B)), 0.0),
                       axis=0, keepdims=True)

    def step(t, carry):
        dmin, cur, ctx, cty, ctz = carry
        sel = li == cur
        cx = jnp.sum(jnp.where(sel, xs, 0.0), axis=1, keepdims=True)
        cy = jnp.sum(jnp.where(sel, ys, 0.0), axis=1, keepdims=True)
        cz = jnp.sum(jnp.where(sel, zs, 0.0), axis=1, keepdims=True)
        hit = kio == t
        ctx = jnp.where(hit, jnp.broadcast_to(transpose_col(cx), (_NP, _B)), ctx)
        cty = jnp.where(hit, jnp.broadcast_to(transpose_col(cy), (_NP, _B)), cty)
        ctz = jnp.where(hit, jnp.broadcast_to(transpose_col(cz), (_NP, _B)), ctz)
        dx = xs - cx
        dy = ys - cy
        dz = zs - cz
        d = (dx * dx + dy * dy) + dz * dz
        dmin = jnp.minimum(dmin, d)
        m = jnp.max(dmin, axis=1, keepdims=True)
        nxt = jnp.min(jnp.where(dmin == m, li, _N), axis=1, keepdims=True)
        return dmin, nxt, ctx, cty, ctz

    init = (
        jnp.full((_B, _N), jnp.inf, jnp.float32),
        jnp.zeros((_B, 1), jnp.int32),
        jnp.zeros((_NP, _B), jnp.float32),
        jnp.zeros((_NP, _B), jnp.float32),
        jnp.zeros((_NP, _B), jnp.float32),
    )
    _, _, ctx, cty, ctz = jax.lax.fori_loop(0, _NP, step, init)
    ctx_ref[...] = ctx
    cty_ref[...] = cty
    ctz_ref[...] = ctz


def _d2_body(xs_ref, ys_ref, zs_ref, cx_ref, cy_ref, cz_ref, d2_ref):
    b = pl.program_id(0)
    xr = xs_ref[0]
    yr = ys_ref[0]
    zr = zs_ref[0]
    lane = jax.lax.broadcasted_iota(jnp.int32, (_NP, _B), 1)
    sel = lane == b

    def col(ref):
        return jnp.sum(jnp.where(sel, ref[...], 0.0), axis=1, keepdims=True)

    acol = col(cx_ref)
    bcol = col(cy_ref)
    ccol = col(cz_ref)
    amat = jnp.concatenate([acol, bcol, ccol], axis=1)
    pmat = jnp.concatenate([xr, yr, zr], axis=0)
    dot = jax.lax.dot_general(
        amat.astype(jnp.bfloat16), pmat.astype(jnp.bfloat16),
        (((1,), (0,)), ((), ())), preferred_element_type=jnp.float32)
    c2 = (acol * acol + bcol * bcol) + ccol * ccol
    p2 = (xr * xr + yr * yr) + zr * zr
    d2_ref[0] = (c2 - 2.0 * dot) + p2


def _topk_body(d2_hbm, out_hbm, rowbuf0, rowbuf1, cvals, cidx, outbuf,
               sem0, sem1):
    wid = lax.axis_index("s") * 2 + lax.axis_index("c")
    lanes = lax.broadcasted_iota(jnp.int32, (_L,), 0)
    lane0 = lanes == 0
    inf_v = jnp.full((_L,), jnp.inf, jnp.float32)
    big_i = jnp.full((_L,), jnp.int32(2 ** 30), jnp.int32)
    base_row = wid * _RPW

    def row_src(rr):
        r = jnp.minimum(base_row + rr, _NROWS - 1)
        return d2_hbm.at[pl.ds(r * _N, _N)]

    def process_row(rr, rowbuf):
        def p1(s, carry):
            mn1, mn2 = carry
            v = rowbuf[pl.ds(s * _L, _L)]
            hi = jnp.maximum(mn1, v)
            mn1 = jnp.minimum(mn1, v)
            mn2 = jnp.minimum(mn2, hi)
            return mn1, mn2

        _, mn2 = lax.fori_loop(0, _STRIPS, p1, (inf_v, inf_v), unroll=16)
        t_scal = jnp.max(mn2)
        t_vec = jnp.full((_L,), t_scal, jnp.float32)

        for t in range(_CAP // _L):
            cvals[pl.ds(t * _L, _L)] = inf_v

        def p2(s, carry):
            pos_vec, base = carry
            v = rowbuf[pl.ds(s * _L, _L)]
            m = v <= t_vec
            okm = m & (pos_vec < _CAP)
            plsc.store_scatter(cvals, [pos_vec], v, mask=okm)
            plsc.store_scatter(cidx, [pos_vec], base, mask=okm)
            pos_vec = pos_vec + (m.astype(jnp.int32) << 4)
            return pos_vec, base + _L

        pos_vec, _ = lax.fori_loop(0, _STRIPS, p2, (lanes, lanes), unroll=16)
        nstrips = jnp.minimum(jnp.max((pos_vec - lanes) >> 4), _CAP // _L)

        def mk_emit(nscan, unroll):
            def emit(j, outpos):
                def scan(s, carry):
                    bv, bi, bp = carry
                    v = cvals[pl.ds(s * _L, _L)]
                    i = cidx[pl.ds(s * _L, _L)]
                    p = s * _L + lanes
                    better = (v < bv) | ((v == bv) & (i < bi))
                    bv = jnp.where(better, v, bv)
                    bi = jnp.where(better, i, bi)
                    bp = jnp.where(better, p, bp)
                    return bv, bi, bp

                bv, bi, bp = lax.fori_loop(
                    0, nscan, scan, (inf_v, big_i, big_i), unroll=unroll)
                mval = jnp.min(bv)
                msel = bv == jnp.full((_L,), mval, jnp.float32)
                midx = jnp.min(jnp.where(msel, bi, big_i))
                midx_vec = jnp.full((_L,), midx, jnp.int32)
                wsel = msel & (bi == midx_vec)
                pw = jnp.min(jnp.where(wsel, bp, big_i))
                pw_vec = jnp.full((_L,), pw, jnp.int32)
                plsc.store_scatter(cvals, [pw_vec], inf_v, mask=lane0)
                plsc.store_scatter(outbuf, [outpos], midx_vec, mask=lane0)
                return outpos + 1

            return emit

        out0 = rr * _PS + jnp.zeros((_L,), jnp.int32)

        def fast_sel():
            lax.fori_loop(0, _PS, mk_emit(16, 4), out0)
            return 0

        def slow_sel():
            lax.fori_loop(0, _PS, mk_emit(nstrips, 1), out0)
            return 0

        lax.cond(nstrips <= 16, fast_sel, slow_sel)

    pltpu.async_copy(row_src(0), rowbuf0, sem0)
    pltpu.async_copy(row_src(1), rowbuf1, sem1)

    def do_pair(i, _):
        rr = 2 * i
        pltpu.make_async_copy(row_src(rr), rowbuf0, sem0).wait()
        process_row(rr, rowbuf0)
        pltpu.async_copy(row_src(rr + 2), rowbuf0, sem0)
        pltpu.make_async_copy(row_src(rr + 1), rowbuf1, sem1).wait()
        process_row(rr + 1, rowbuf1)
        pltpu.async_copy(row_src(rr + 3), rowbuf1, sem1)
        return 0

    lax.fori_loop(0, _RPW // 2, do_pair, 0)
    pltpu.make_async_copy(row_src(0), rowbuf0, sem0).wait()
    pltpu.make_async_copy(row_src(1), rowbuf1, sem1).wait()
    pltpu.sync_copy(outbuf, out_hbm.at[pl.ds(wid * (_RPW * _PS), _RPW * _PS)])


def _sc_topk(d2_flat):
    mesh = plsc.VectorSubcoreMesh(core_axis_name="c", subcore_axis_name="s")
    kfn = functools.partial(
        pl.kernel,
        out_type=jax.ShapeDtypeStruct((_NROWS * _PS,), jnp.int32),
        mesh=mesh,
        scratch_types=[
            pltpu.VMEM((_N,), jnp.float32),
            pltpu.VMEM((_N,), jnp.float32),
            pltpu.VMEM((_CAP,), jnp.float32),
            pltpu.VMEM((_CAP,), jnp.int32),
            pltpu.VMEM((_RPW * _PS,), jnp.int32),
            pltpu.SemaphoreType.DMA,
            pltpu.SemaphoreType.DMA,
        ],
        compiler_params=pltpu.CompilerParams(needs_layout_passes=False),
    )(_topk_body)
    return kfn(d2_flat)


def kernel(points):
    B, N, _ = points.shape
    pts = jnp.transpose(points, (0, 2, 1))
    xs = pts[:, 0, :]
    ys = pts[:, 1, :]
    zs = pts[:, 2, :]

    cen_shape = jax.ShapeDtypeStruct((_NP, _B), jnp.float32)
    ctx, cty, ctz = pl.pallas_call(
        _fps_body,
        out_shape=[cen_shape, cen_shape, cen_shape],
    )(xs, ys, zs)

    xs3 = xs.reshape(B, 1, N)
    ys3 = ys.reshape(B, 1, N)
    zs3 = zs.reshape(B, 1, N)
    row_spec = pl.BlockSpec((1, 1, _N), lambda b: (b, 0, 0))
    cen_spec = pl.BlockSpec((_NP, _B), lambda b: (0, 0))
    d2 = pl.pallas_call(
        _d2_body,
        grid=(B,),
        in_specs=[row_spec, row_spec, row_spec, cen_spec, cen_spec, cen_spec],
        out_specs=pl.BlockSpec((1, _NP, _N), lambda b: (b, 0, 0)),
        out_shape=jax.ShapeDtypeStruct((B, _NP, _N), jnp.float32),
    )(xs3, ys3, zs3, ctx, cty, ctz)

    centers = jnp.transpose(jnp.stack([ctx, cty, ctz], axis=0), (2, 1, 0))
    idx = _sc_topk(d2.reshape(-1)).reshape(B, _NP, _PS)
    return idx, centers
